# Initial kernel scaffold; baseline (speedup 1.0000x reference)
import jax
import jax.numpy as jnp
from jax.experimental import pallas as pl


def kernel(x, Wr, sW1, sb1, sW2, sb2, pW1, pb1, pW2, pb2):
    raise NotImplementedError("write your pallas kernel here")


# speedup vs baseline: 2.8191x; 2.8191x over previous
import jax
import jax.numpy as jnp
from jax.experimental import pallas as pl
from jax.experimental.pallas import tpu as pltpu

B, S, D = 1, 2048, 768
FF = 3072
E = 8
TT = 256
FH = FF // 2


def _gelu(v):
    return 0.5 * v * (1.0 + jax.lax.erf(v * 0.7071067811865476))


def _router_shared_body(x_ref, wr_ref, w1_ref, b1_ref, w2_ref, b2_ref,
                        out_ref, mask_ref):
    xt = x_ref[:]
    logits = jnp.dot(xt.astype(jnp.bfloat16), wr_ref[:].astype(jnp.bfloat16),
                     preferred_element_type=jnp.float32)
    m = jnp.max(logits, axis=1, keepdims=True)
    p = jnp.exp(logits - m)
    p = p / jnp.sum(p, axis=1, keepdims=True)
    lane = jax.lax.broadcasted_iota(jnp.int32, p.shape, 1)
    v1 = jnp.max(p, axis=1, keepdims=True)
    i1 = jnp.min(jnp.where(p == v1, lane, E), axis=1, keepdims=True)
    p2 = jnp.where(lane == i1, -1.0, p)
    v2 = jnp.max(p2, axis=1, keepdims=True)
    i2 = jnp.min(jnp.where(p2 == v2, lane, E), axis=1, keepdims=True)
    keep = (lane == i1) | (lane == i2)
    mask_ref[:] = jnp.where(keep, p, 0.0) / (v1 + v2)

    xb = xt.astype(jnp.bfloat16)
    h = jnp.dot(xb, w1_ref[:].astype(jnp.bfloat16),
                preferred_element_type=jnp.float32) + b1_ref[:]
    h = _gelu(h)
    out_ref[:] = jnp.dot(h.astype(jnp.bfloat16), w2_ref[:].astype(jnp.bfloat16),
                         preferred_element_type=jnp.float32) + b2_ref[:]


def _experts_body(x_ref, w1_ref, b1_ref, w2_ref, b2_ref, mask_ref, shared_ref,
                  out_ref, acc_ref):
    e = pl.program_id(0)
    f = pl.program_id(1)
    t = pl.program_id(2)
    base = t * TT
    xt = x_ref[pl.ds(base, TT), :].astype(jnp.bfloat16)
    h = jnp.dot(xt, w1_ref[0].astype(jnp.bfloat16),
                preferred_element_type=jnp.float32) + b1_ref[0, 0]
    h = _gelu(h)
    o = jnp.dot(h.astype(jnp.bfloat16), w2_ref[0].astype(jnp.bfloat16),
                preferred_element_type=jnp.float32)

    wfull = mask_ref[pl.ds(base, TT), :]
    lane = jax.lax.broadcasted_iota(jnp.int32, wfull.shape, 1)
    w = jnp.sum(jnp.where(lane == e, wfull, 0.0), axis=1, keepdims=True)

    contrib = w * o

    @pl.when((e == 0) & (f == 0))
    def _():
        acc_ref[pl.ds(base, TT), :] = shared_ref[:] + contrib + w * b2_ref[0]

    @pl.when((e > 0) & (f == 0))
    def _():
        acc_ref[pl.ds(base, TT), :] = (acc_ref[pl.ds(base, TT), :]
                                       + contrib + w * b2_ref[0])

    @pl.when(f == 1)
    def _():
        acc_ref[pl.ds(base, TT), :] = acc_ref[pl.ds(base, TT), :] + contrib

    out_ref[:] = acc_ref[pl.ds(base, TT), :]


def kernel(x, Wr, sW1, sb1, sW2, sb2, pW1, pb1, pW2, pb2):
    xs = x.reshape(S, D)

    shared_out, mask = pl.pallas_call(
        _router_shared_body,
        grid=(S // TT,),
        in_specs=[
            pl.BlockSpec((TT, D), lambda t: (t, 0)),
            pl.BlockSpec((D, E), lambda t: (0, 0)),
            pl.BlockSpec((D, FF), lambda t: (0, 0)),
            pl.BlockSpec((1, FF), lambda t: (0, 0)),
            pl.BlockSpec((FF, D), lambda t: (0, 0)),
            pl.BlockSpec((1, D), lambda t: (0, 0)),
        ],
        out_specs=[
            pl.BlockSpec((TT, D), lambda t: (t, 0)),
            pl.BlockSpec((TT, E), lambda t: (t, 0)),
        ],
        out_shape=[
            jax.ShapeDtypeStruct((S, D), jnp.float32),
            jax.ShapeDtypeStruct((S, E), jnp.float32),
        ],
    )(xs, Wr, sW1, sb1.reshape(1, FF), sW2, sb2.reshape(1, D))

    routed = pl.pallas_call(
        _experts_body,
        grid=(E, 2, S // TT),
        in_specs=[
            pl.BlockSpec((S, D), lambda e, f, t: (0, 0)),
            pl.BlockSpec((1, D, FH), lambda e, f, t: (e, 0, f)),
            pl.BlockSpec((1, 1, FH), lambda e, f, t: (2 * e + f, 0, 0)),
            pl.BlockSpec((1, FH, D), lambda e, f, t: (e, f, 0)),
            pl.BlockSpec((1, 1, D), lambda e, f, t: (e, 0, 0)),
            pl.BlockSpec((S, E), lambda e, f, t: (0, 0)),
            pl.BlockSpec((TT, D),
                         lambda e, f, t: (jnp.where((e == 0) & (f == 0),
                                                    t, S // TT - 1), 0)),
        ],
        out_specs=pl.BlockSpec((TT, D), lambda e, f, t: (t, 0)),
        out_shape=jax.ShapeDtypeStruct((S, D), jnp.float32),
        scratch_shapes=[pltpu.VMEM((S, D), jnp.float32)],
    )(xs, pW1, pb1.reshape(E * 2, 1, FH), pW2, pb2.reshape(E, 1, D), mask,
      shared_out)

    return routed.reshape(B, S, D)
